# baseline (speedup 1.0000x reference)
import jax
import jax.numpy as jnp
from jax import lax
from jax.experimental import pallas as pl
from jax.experimental.pallas import tpu as pltpu
from jax.experimental.pallas import tpu_sc as plsc

N_NODES = 10000
N_PAD = 10240
D_IN = 128
D_HID = 64
W72 = 72
W8 = 8
NC = 2
NS = 16
NW = NC * NS
CH = 128
BM = 1000
RPT = N_PAD // NS
TRPT = N_NODES // NS

_mesh = plsc.VectorSubcoreMesh(core_axis_name="c", subcore_axis_name="s")


def _seg_kernel(n_grp, d, nbuf):
    assert n_grp % nbuf == 0

    def body3(y_hbm, src_hbm, dst_hbm, z_hbm, acc_out,
              src_v, dst_v, r0, r1, r2, tbl_sh, acc_sh,
              gs0, gs1, gs2, ss0, ss1, ss2):
        c = lax.axis_index("c")
        s = lax.axis_index("s")
        wid = c * NS + s
        pltpu.sync_copy(src_hbm.at[wid], src_v)
        pltpu.sync_copy(dst_hbm.at[wid], dst_v)
        pltpu.sync_copy(y_hbm.at[pl.ds(s * TRPT, TRPT)],
                        tbl_sh.at[pl.ds(s * TRPT, TRPT)])
        pltpu.sync_copy(z_hbm.at[pl.ds(s * RPT, RPT)],
                        acc_sh.at[pl.ds(s * RPT, RPT)])
        plsc.subcore_barrier()

        bufs = (r0, r1, r2)
        gsems = (gs0, gs1, gs2)
        ssems = (ss0, ss1, ss2)

        def gather(j, k):
            return pltpu.async_copy(
                tbl_sh.at[src_v.at[lax.min(j, n_grp - 1)]], bufs[k], gsems[k])

        gather(0, 0)
        gather(1, 1)

        def phase(j, k, kp):
            pltpu.make_async_copy(
                tbl_sh.at[src_v.at[lax.min(j, n_grp - 1)]],
                bufs[k], gsems[k]).wait()
            pltpu.async_copy(bufs[k], acc_sh.at[dst_v.at[j]], ssems[k],
                             add=True)

            @pl.when(j >= 1)
            def _():
                pltpu.make_async_copy(
                    bufs[kp], acc_sh.at[dst_v.at[lax.max(j - 1, 0)]],
                    ssems[kp]).wait()
            gather(j + 2, kp)

        def triple(jj, carry):
            j = jj * 3
            phase(j, 0, 2)
            phase(j + 1, 1, 0)
            phase(j + 2, 2, 1)
            return carry

        lax.fori_loop(0, n_grp // 3, triple, 0)
        pltpu.make_async_copy(bufs[2], acc_sh.at[dst_v.at[n_grp - 1]],
                              ssems[2]).wait()
        pltpu.make_async_copy(tbl_sh.at[src_v.at[n_grp - 1]], bufs[0],
                              gsems[0]).wait()
        pltpu.make_async_copy(tbl_sh.at[src_v.at[n_grp - 1]], bufs[1],
                              gsems[1]).wait()
        plsc.subcore_barrier()
        pltpu.sync_copy(acc_sh.at[pl.ds(s * RPT, RPT)],
                        acc_out.at[c, pl.ds(s * RPT, RPT)])

    def body2(y_hbm, src_hbm, dst_hbm, z_hbm, acc_out,
              src_v, dst_v, rows_a, rows_b, tbl_sh, acc_sh, gsem_a, gsem_b):
        c = lax.axis_index("c")
        s = lax.axis_index("s")
        wid = c * NS + s
        pltpu.sync_copy(src_hbm.at[wid], src_v)
        pltpu.sync_copy(dst_hbm.at[wid], dst_v)
        pltpu.sync_copy(y_hbm.at[pl.ds(s * TRPT, TRPT)],
                        tbl_sh.at[pl.ds(s * TRPT, TRPT)])
        pltpu.sync_copy(z_hbm.at[pl.ds(s * RPT, RPT)],
                        acc_sh.at[pl.ds(s * RPT, RPT)])
        plsc.subcore_barrier()

        def gather(j, buf, sem):
            return pltpu.async_copy(
                tbl_sh.at[src_v.at[lax.min(j, n_grp - 1)]], buf, sem)

        gather(0, rows_a, gsem_a)

        def phase(j, buf, gsem, nxt_buf, nxt_gsem):
            pltpu.make_async_copy(
                tbl_sh.at[src_v.at[lax.min(j, n_grp - 1)]], buf, gsem).wait()
            gather(j + 1, nxt_buf, nxt_gsem)
            pltpu.sync_copy(buf, acc_sh.at[dst_v.at[j]], add=True)

        def pair(jj, carry):
            j = jj * 2
            phase(j, rows_a, gsem_a, rows_b, gsem_b)
            phase(j + 1, rows_b, gsem_b, rows_a, gsem_a)
            return carry

        lax.fori_loop(0, n_grp // 2, pair, 0)
        pltpu.make_async_copy(tbl_sh.at[src_v.at[n_grp - 1]], rows_a,
                              gsem_a).wait()
        plsc.subcore_barrier()
        pltpu.sync_copy(acc_sh.at[pl.ds(s * RPT, RPT)],
                        acc_out.at[c, pl.ds(s * RPT, RPT)])

    scratch = [
        pltpu.VMEM((n_grp, CH), jnp.int32),
        pltpu.VMEM((n_grp, CH), jnp.int32),
    ]
    scratch += [pltpu.VMEM((CH, d), jnp.float32)] * nbuf
    scratch += [
        pltpu.VMEM_SHARED((N_NODES, d), jnp.float32),
        pltpu.VMEM_SHARED((N_PAD, d), jnp.float32),
    ]
    scratch += [pltpu.SemaphoreType.DMA] * nbuf
    if nbuf == 3:
        scratch += [pltpu.SemaphoreType.DMA] * nbuf
    return pl.kernel(
        body3 if nbuf == 3 else body2,
        out_type=jax.ShapeDtypeStruct((NC, N_PAD, d), jnp.float32),
        mesh=_mesh,
        scratch_types=scratch,
        compiler_params=pltpu.CompilerParams(use_tc_tiling_on_sc=False),
    )


def _mm0_call():
    def body(x_ref, w_ref, o_ref):
        y = jnp.dot(x_ref[...], w_ref[...], preferred_element_type=jnp.float32)
        pad = jnp.concatenate(
            [jnp.ones((BM, 1), jnp.float32),
             jnp.zeros((BM, W72 - D_HID - 1), jnp.float32)], axis=1)
        o_ref[...] = jnp.concatenate([y, pad], axis=1)
    return pl.pallas_call(
        body,
        grid=(N_NODES // BM,),
        in_specs=[pl.BlockSpec((BM, D_IN), lambda i: (i, 0)),
                  pl.BlockSpec((D_IN, D_HID), lambda i: (0, 0))],
        out_specs=pl.BlockSpec((BM, W72), lambda i: (i, 0)),
        out_shape=jax.ShapeDtypeStruct((N_PAD, W72), jnp.float32),
    )


def _upd_call(acc_w, hp_w, out_w, next_w, final, first):
    def body(*refs):
        if first:
            (a0, a1, hp, wr, b), rest = refs[:5], refs[5:]
        else:
            (a0, a1, inv_in, hp, wr, b), rest = refs[:6], refs[6:]
        if next_w is not None:
            wl = rest[0]
            outs = rest[1:]
        else:
            outs = rest
        a = a0[0] + a1[0]
        if first:
            cnt = a[:, D_HID:D_HID + 1]
            agg = a[:, :D_HID]
            inv = 1.0 / jnp.maximum(cnt, 1.0)
        else:
            inv = inv_in[...]
            agg = a
        z = agg * inv + jnp.dot(hp[...], wr[...],
                                preferred_element_type=jnp.float32) + b[...]
        if final:
            h = 1.0 / (1.0 + jnp.exp(-z))
        else:
            h = jnp.maximum(z, 0.0)
        if final:
            outs[0][...] = h[:, 0:1]
        else:
            outs[0][...] = h
        if next_w is not None:
            outs[1][...] = jnp.dot(h, wl[...],
                                   preferred_element_type=jnp.float32)
        if first:
            outs[2][...] = inv

    in_specs = [pl.BlockSpec((1, BM, acc_w), lambda i: (0, i, 0)),
                pl.BlockSpec((1, BM, acc_w), lambda i: (1, i, 0))]
    if not first:
        in_specs.append(pl.BlockSpec((BM, 1), lambda i: (i, 0)))
    in_specs += [pl.BlockSpec((BM, hp_w), lambda i: (i, 0)),
                 pl.BlockSpec((hp_w, out_w), lambda i: (0, 0)),
                 pl.BlockSpec((1, out_w), lambda i: (0, 0))]
    h_w = 1 if final else out_w
    out_shape = [jax.ShapeDtypeStruct((N_NODES, h_w), jnp.float32)]
    out_specs = [pl.BlockSpec((BM, h_w), lambda i: (i, 0))]
    if next_w is not None:
        in_specs.append(pl.BlockSpec((out_w, next_w), lambda i: (0, 0)))
        out_shape.append(jax.ShapeDtypeStruct((N_PAD, next_w), jnp.float32))
        out_specs.append(pl.BlockSpec((BM, next_w), lambda i: (i, 0)))
    if first:
        out_shape.append(jax.ShapeDtypeStruct((N_NODES, 1), jnp.float32))
        out_specs.append(pl.BlockSpec((BM, 1), lambda i: (i, 0)))
    f = pl.pallas_call(
        body, grid=(N_NODES // BM,),
        in_specs=in_specs, out_specs=out_specs, out_shape=out_shape,
    )
    if next_w is None:
        return lambda *a: f(*a)[0]
    return f


def kernel(x, edge_index, Wl1, Wr1, b1, Wl2, Wr2, b2, Wl3, Wr3, b3):
    e = edge_index.shape[1]
    n_grp = -(-e // (NW * CH))
    n_grp = -(-n_grp // 6) * 6
    e_pad = NW * n_grp * CH
    src = edge_index[0].astype(jnp.int32)
    dst = edge_index[1].astype(jnp.int32)
    src = jnp.concatenate([src, jnp.zeros((e_pad - e,), jnp.int32)])
    dst = jnp.concatenate([dst, jnp.full((e_pad - e,), N_PAD - 1, jnp.int32)])
    srcw = src.reshape(NW, n_grp, CH)
    dstw = dst.reshape(NW, n_grp, CH)

    z72 = jnp.zeros((N_PAD, W72), jnp.float32)
    z64 = jnp.zeros((N_PAD, D_HID), jnp.float32)
    z8 = jnp.zeros((N_PAD, W8), jnp.float32)
    Wl3p = jnp.pad(Wl3, ((0, 0), (0, W8 - 1)))
    Wr3p = jnp.pad(Wr3, ((0, 0), (0, W8 - 1)))
    b3p = jnp.pad(b3, (0, W8 - 1)).reshape(1, W8)

    y1 = _mm0_call()(x, Wl1)
    acc1 = _seg_kernel(n_grp, W72, 2)(y1, srcw, dstw, z72)
    h1, y2, inv = _upd_call(W72, D_IN, D_HID, D_HID, final=False, first=True)(
        acc1, acc1, x, Wr1, b1.reshape(1, D_HID), Wl2)
    acc2 = _seg_kernel(n_grp, D_HID, 3)(y2, srcw, dstw, z64)
    h2, y3 = _upd_call(D_HID, D_HID, D_HID, W8, final=False, first=False)(
        acc2, acc2, inv, h1, Wr2, b2.reshape(1, D_HID), Wl3p)
    acc3 = _seg_kernel(n_grp, W8, 3)(y3, srcw, dstw, z8)
    return _upd_call(W8, D_HID, W8, None, final=True, first=False)(
        acc3, acc3, inv, h2, Wr3p, b3p)


# speedup vs baseline: 1.0375x; 1.0375x over previous
import jax
import jax.numpy as jnp
from jax import lax
from jax.experimental import pallas as pl
from jax.experimental.pallas import tpu as pltpu
from jax.experimental.pallas import tpu_sc as plsc

N_NODES = 10000
N_PAD = 10240
D_IN = 128
D_HID = 64
W72 = 72
W8 = 8
NC = 2
NS = 16
NW = NC * NS
CH = 128
BM = 1000
RPT = N_PAD // NS
TRPT = N_NODES // NS

_mesh = plsc.VectorSubcoreMesh(core_axis_name="c", subcore_axis_name="s")


def _seg_kernel(n_grp, d):
    assert n_grp % 2 == 0

    def body2(y_hbm, src_hbm, dst_hbm, z_hbm, acc_out,
              src_v, dst_v, rows_a, rows_b, tbl_sh, acc_sh, gsem_a, gsem_b):
        c = lax.axis_index("c")
        s = lax.axis_index("s")
        wid = c * NS + s
        pltpu.sync_copy(src_hbm.at[wid], src_v)
        pltpu.sync_copy(dst_hbm.at[wid], dst_v)
        pltpu.sync_copy(y_hbm.at[pl.ds(s * TRPT, TRPT)],
                        tbl_sh.at[pl.ds(s * TRPT, TRPT)])
        pltpu.sync_copy(z_hbm.at[pl.ds(s * RPT, RPT)],
                        acc_sh.at[pl.ds(s * RPT, RPT)])
        plsc.subcore_barrier()

        def gather(j, buf, sem):
            return pltpu.async_copy(
                tbl_sh.at[src_v.at[lax.min(j, n_grp - 1)]], buf, sem)

        gather(0, rows_a, gsem_a)

        def phase(j, buf, gsem, nxt_buf, nxt_gsem):
            pltpu.make_async_copy(
                tbl_sh.at[src_v.at[lax.min(j, n_grp - 1)]], buf, gsem).wait()
            gather(j + 1, nxt_buf, nxt_gsem)
            pltpu.sync_copy(buf, acc_sh.at[dst_v.at[j]], add=True)

        def pair(jj, carry):
            j = jj * 2
            phase(j, rows_a, gsem_a, rows_b, gsem_b)
            phase(j + 1, rows_b, gsem_b, rows_a, gsem_a)
            return carry

        lax.fori_loop(0, n_grp // 2, pair, 0)
        pltpu.make_async_copy(tbl_sh.at[src_v.at[n_grp - 1]], rows_a,
                              gsem_a).wait()
        plsc.subcore_barrier()
        pltpu.sync_copy(acc_sh.at[pl.ds(s * RPT, RPT)],
                        acc_out.at[c, pl.ds(s * RPT, RPT)])

    scratch = [
        pltpu.VMEM((n_grp, CH), jnp.int32),
        pltpu.VMEM((n_grp, CH), jnp.int32),
    ]
    scratch += [pltpu.VMEM((CH, d), jnp.float32)] * 2
    scratch += [
        pltpu.VMEM_SHARED((N_NODES, d), jnp.float32),
        pltpu.VMEM_SHARED((N_PAD, d), jnp.float32),
    ]
    scratch += [pltpu.SemaphoreType.DMA] * 2
    return pl.kernel(
        body2,
        out_type=jax.ShapeDtypeStruct((NC, N_PAD, d), jnp.float32),
        mesh=_mesh,
        scratch_types=scratch,
        compiler_params=pltpu.CompilerParams(use_tc_tiling_on_sc=False),
    )


def _mm0_call():
    def body(x_ref, w_ref, o_ref):
        y = jnp.dot(x_ref[...], w_ref[...], preferred_element_type=jnp.float32)
        pad = jnp.concatenate(
            [jnp.ones((BM, 1), jnp.float32),
             jnp.zeros((BM, W72 - D_HID - 1), jnp.float32)], axis=1)
        o_ref[...] = jnp.concatenate([y, pad], axis=1)
    return pl.pallas_call(
        body,
        grid=(N_NODES // BM,),
        in_specs=[pl.BlockSpec((BM, D_IN), lambda i: (i, 0)),
                  pl.BlockSpec((D_IN, D_HID), lambda i: (0, 0))],
        out_specs=pl.BlockSpec((BM, W72), lambda i: (i, 0)),
        out_shape=jax.ShapeDtypeStruct((N_PAD, W72), jnp.float32),
    )


def _upd_call(acc_w, hp_w, out_w, next_w, final, first):
    def body(*refs):
        if first:
            (a0, a1, hp, wr, b), rest = refs[:5], refs[5:]
        else:
            (a0, a1, inv_in, hp, wr, b), rest = refs[:6], refs[6:]
        if next_w is not None:
            wl = rest[0]
            outs = rest[1:]
        else:
            outs = rest
        a = a0[0] + a1[0]
        if first:
            cnt = a[:, D_HID:D_HID + 1]
            agg = a[:, :D_HID]
            inv = 1.0 / jnp.maximum(cnt, 1.0)
        else:
            inv = inv_in[...]
            agg = a
        z = agg * inv + jnp.dot(hp[...], wr[...],
                                preferred_element_type=jnp.float32) + b[...]
        if final:
            h = 1.0 / (1.0 + jnp.exp(-z))
        else:
            h = jnp.maximum(z, 0.0)
        if final:
            outs[0][...] = h[:, 0:1]
        else:
            outs[0][...] = h
        if next_w is not None:
            outs[1][...] = jnp.dot(h, wl[...],
                                   preferred_element_type=jnp.float32)
        if first:
            outs[2][...] = inv

    in_specs = [pl.BlockSpec((1, BM, acc_w), lambda i: (0, i, 0)),
                pl.BlockSpec((1, BM, acc_w), lambda i: (1, i, 0))]
    if not first:
        in_specs.append(pl.BlockSpec((BM, 1), lambda i: (i, 0)))
    in_specs += [pl.BlockSpec((BM, hp_w), lambda i: (i, 0)),
                 pl.BlockSpec((hp_w, out_w), lambda i: (0, 0)),
                 pl.BlockSpec((1, out_w), lambda i: (0, 0))]
    h_w = 1 if final else out_w
    out_shape = [jax.ShapeDtypeStruct((N_NODES, h_w), jnp.float32)]
    out_specs = [pl.BlockSpec((BM, h_w), lambda i: (i, 0))]
    if next_w is not None:
        in_specs.append(pl.BlockSpec((out_w, next_w), lambda i: (0, 0)))
        out_shape.append(jax.ShapeDtypeStruct((N_PAD, next_w), jnp.float32))
        out_specs.append(pl.BlockSpec((BM, next_w), lambda i: (i, 0)))
    if first:
        out_shape.append(jax.ShapeDtypeStruct((N_NODES, 1), jnp.float32))
        out_specs.append(pl.BlockSpec((BM, 1), lambda i: (i, 0)))
    f = pl.pallas_call(
        body, grid=(N_NODES // BM,),
        in_specs=in_specs, out_specs=out_specs, out_shape=out_shape,
    )
    if next_w is None:
        return lambda *a: f(*a)[0]
    return f


def kernel(x, edge_index, Wl1, Wr1, b1, Wl2, Wr2, b2, Wl3, Wr3, b3):
    e = edge_index.shape[1]
    n_grp = -(-e // (NW * CH))
    n_grp += n_grp % 2
    e_pad = NW * n_grp * CH
    src = edge_index[0].astype(jnp.int32)
    dst = edge_index[1].astype(jnp.int32)
    src = jnp.concatenate([src, jnp.zeros((e_pad - e,), jnp.int32)])
    dst = jnp.concatenate([dst, jnp.full((e_pad - e,), N_PAD - 1, jnp.int32)])
    srcw = src.reshape(NW, n_grp, CH)
    dstw = dst.reshape(NW, n_grp, CH)

    z72 = jnp.zeros((N_PAD, W72), jnp.float32)
    z64 = jnp.zeros((N_PAD, D_HID), jnp.float32)
    z8 = jnp.zeros((N_PAD, W8), jnp.float32)
    Wl3p = jnp.pad(Wl3, ((0, 0), (0, W8 - 1)))
    Wr3p = jnp.pad(Wr3, ((0, 0), (0, W8 - 1)))
    b3p = jnp.pad(b3, (0, W8 - 1)).reshape(1, W8)

    y1 = _mm0_call()(x, Wl1)
    acc1 = _seg_kernel(n_grp, W72)(y1, srcw, dstw, z72)
    h1, y2, inv = _upd_call(W72, D_IN, D_HID, D_HID, final=False, first=True)(
        acc1, acc1, x, Wr1, b1.reshape(1, D_HID), Wl2)
    acc2 = _seg_kernel(n_grp, D_HID)(y2, srcw, dstw, z64)
    h2, y3 = _upd_call(D_HID, D_HID, D_HID, W8, final=False, first=False)(
        acc2, acc2, inv, h1, Wr2, b2.reshape(1, D_HID), Wl3p)
    acc3 = _seg_kernel(n_grp, W8)(y3, srcw, dstw, z8)
    return _upd_call(W8, D_HID, W8, None, final=True, first=False)(
        acc3, acc3, inv, h2, Wr3p, b3p)
